# baseline (speedup 1.0000x reference)
import dataclasses
import functools

import jax
import jax.numpy as jnp
import numpy as np
from jax import lax
from jax.experimental import pallas as pl
from jax.experimental.pallas import tpu as pltpu
from jax.experimental.pallas import tpu_sc as plsc

_K = 8
_INV = float(1.0 / np.sqrt(1.0 + 1e-5))

_NC = 2
_NS = 16
_L = 16


def _sc_transpose(embT, tail_flat, V, D):
    CK = 1024
    n_full = V // CK
    tail = V - n_full * CK
    rounds = (n_full + _NC * _NS - 1) // (_NC * _NS)

    mesh = plsc.VectorSubcoreMesh(core_axis_name="c", subcore_axis_name="s")
    cp = pltpu.CompilerParams(needs_layout_passes=False,
                              use_tc_tiling_on_sc=True)

    nw = _NC * _NS
    assert rounds % 2 == 0

    @functools.partial(
        pl.kernel,
        mesh=mesh,
        compiler_params=cp,
        out_type=jax.ShapeDtypeStruct((V * D,), jnp.float32),
        scratch_types=[
            pltpu.VMEM((D, CK + 1), jnp.float32),
            pltpu.VMEM((D, CK + 1), jnp.float32),
            pltpu.VMEM((CK * D,), jnp.float32),
            pltpu.VMEM((CK * D,), jnp.float32),
            pltpu.SemaphoreType.DMA,
            pltpu.SemaphoreType.DMA,
            pltpu.SemaphoreType.DMA,
            pltpu.SemaphoreType.DMA,
        ],
    )
    def transpose_kernel(embT_hbm, out_hbm, buf0, buf1, outf0, outf1,
                         si0, si1, so0, so1):
        wid = lax.axis_index("s") * _NC + lax.axis_index("c")
        diota = lax.iota(jnp.int32, _L)
        zeros = jnp.zeros((_L,), jnp.int32)

        def in_desc(t, buf, sem):
            c0 = (t * nw + wid) * CK
            return pltpu.make_async_copy(
                embT_hbm.at[:, pl.ds(c0, CK)], buf.at[:, pl.ds(0, CK)], sem)

        def out_desc(t, outf, sem):
            c0 = (t * nw + wid) * CK
            return pltpu.make_async_copy(
                outf, out_hbm.at[pl.ds(c0 * D, CK * D)], sem)

        def valid(t):
            return (t >= 0) & (t < rounds) & (t * nw + wid < n_full)

        def start_in(t, buf, sem):
            @pl.when(valid(t))
            def _():
                in_desc(t, buf, sem).start()

        def step(t, buf, sem_i, outf, sem_o):
            @pl.when(valid(t - 2))
            def _():
                out_desc(t - 2, outf, sem_o).wait()

            @pl.when(valid(t))
            def _():
                in_desc(t, buf, sem_i).wait()

                @pl.loop(0, CK, step=4)
                def _cols(l0):
                    for h in range(4):
                        lo = l0 + h
                        vals = plsc.load_gather(buf, [diota, zeros + lo])
                        outf[pl.ds(lo * D, _L)] = vals

                out_desc(t, outf, sem_o).start()

        start_in(0, buf0, si0)
        start_in(1, buf1, si1)

        @pl.loop(0, rounds, step=2)
        def _round(t):
            start_in(t + 2, buf0, si0)
            step(t, buf0, si0, outf0, so0)
            start_in(t + 3, buf1, si1)
            step(t + 1, buf1, si1, outf1, so1)

        @pl.when(valid(rounds - 2))
        def _():
            out_desc(rounds - 2, outf0, so0).wait()

        @pl.when(valid(rounds - 1))
        def _():
            out_desc(rounds - 1, outf1, so1).wait()

    out = transpose_kernel(embT)
    if tail:
        out = lax.dynamic_update_slice(out, tail_flat, (n_full * CK * D,))
    return out


def _sc_gather(emb, lin2, idx_flat):
    n_idx = idx_flat.shape[0]
    V, D = emb.shape
    nw = _NC * _NS
    per_w = n_idx // nw
    C = 1664
    nchunk = per_w // C
    assert per_w % C == 0

    mesh = plsc.VectorSubcoreMesh(core_axis_name="c", subcore_axis_name="s")
    cp = pltpu.CompilerParams(needs_layout_passes=False,
                              use_tc_tiling_on_sc=False)

    @functools.partial(
        pl.kernel,
        mesh=mesh,
        compiler_params=cp,
        out_type=[
            jax.ShapeDtypeStruct((n_idx, D), jnp.float32),
            jax.ShapeDtypeStruct((n_idx,), jnp.float32),
        ],
        scratch_types=[
            pltpu.VMEM((C,), jnp.int32),
            pltpu.VMEM((C,), jnp.int32),
            pltpu.VMEM((C,), jnp.int32),
            pltpu.VMEM((C, D), jnp.float32),
            pltpu.VMEM((C, _L), jnp.float32),
            pltpu.VMEM((C,), jnp.float32),
            pltpu.SemaphoreType.DMA,
            pltpu.SemaphoreType.DMA,
        ],
    )
    def gather_kernel(emb_hbm, lin_hbm, idx_hbm, eg_hbm, lv_hbm,
                      idx_v, hi_v, lo_v, erows_v, lgran_v, lv_v, sem0, sem1):
        wid = lax.axis_index("s") * _NC + lax.axis_index("c")
        base = wid * per_w

        @pl.loop(0, nchunk)
        def _chunk(k):
            off = base + k * C
            pltpu.sync_copy(idx_hbm.at[pl.ds(off, C)], idx_v)

            @pl.loop(0, C, step=_L)
            def _split(i):
                v = idx_v[pl.ds(i, _L)]
                hi_v[pl.ds(i, _L)] = lax.shift_right_logical(v, 4)
                lo_v[pl.ds(i, _L)] = lax.bitwise_and(v, 15)

            ce = pltpu.async_copy(emb_hbm.at[idx_v], erows_v, sem0)
            cl = pltpu.async_copy(lin_hbm.at[hi_v], lgran_v, sem1)
            ce.wait()
            cl.wait()

            @pl.loop(0, C, step=_L)
            def _select(i):
                rows = lax.iota(jnp.int32, _L) + i
                lv_v[pl.ds(i, _L)] = plsc.load_gather(
                    lgran_v, [rows, lo_v[pl.ds(i, _L)]])

            pltpu.sync_copy(erows_v, eg_hbm.at[pl.ds(off, C)])
            pltpu.sync_copy(lv_v, lv_hbm.at[pl.ds(off, C)])

    return gather_kernel(emb, lin2, idx_flat)


def _dense_body(F, eg_ref, lv_ref, sf_ref, tf_ref, cw_ref, ca_ref, cc_ref,
                E_ref, w1_ref, a1_ref, c1_ref, w2_ref, a2_ref, c2_ref,
                w3_ref, cb_ref, y_ref):
    hi = lax.Precision.DEFAULT
    eb = eg_ref[...] * sf_ref[...] + tf_ref[...]
    z = jnp.dot(eb, cw_ref[...], precision=hi,
                preferred_element_type=jnp.float32)
    u = jnp.maximum(z * ca_ref[...] + cc_ref[...], 0.0)

    ut = u.T
    ii = lax.broadcasted_iota(jnp.int32, ut.shape, 0)
    rank = jnp.zeros(ut.shape, jnp.int32)
    for j in range(F):
        uj = ut[j:j + 1, :]
        beats = (uj > ut) | ((uj == ut) & (j < ii))
        rank = rank + beats.astype(jnp.int32)
    sel = rank < _K

    m = jnp.max(ut, axis=0, keepdims=True)
    p = jnp.where(sel, jnp.exp(ut - m), 0.0)
    s = jnp.sum(p, axis=0, keepdims=True)
    mask = (p / s).T

    me = jnp.dot(mask, E_ref[...], precision=hi,
                 preferred_element_type=jnp.float32)
    em = eb * me
    h1 = jnp.maximum(jnp.dot(em, w1_ref[...], precision=hi,
                             preferred_element_type=jnp.float32)
                     * a1_ref[...] + c1_ref[...], 0.0)
    h2 = jnp.maximum(jnp.dot(h1, w2_ref[...], precision=hi,
                             preferred_element_type=jnp.float32)
                     * a2_ref[...] + c2_ref[...], 0.0)
    out = jnp.sum(h2 * w3_ref[...], axis=1, keepdims=True)
    lin = jnp.sum(lv_ref[...], axis=1, keepdims=True)
    y_ref[...] = jax.nn.sigmoid(out + lin + cb_ref[0, 0]).T


def _tc_dense(eg2, lv2, sf, tf, cw, ca, cc, E, w1, a1, c1, w2, a2, c2, w3t, cb):
    B, FD = eg2.shape
    F = lv2.shape[1]
    R = 512
    grid = (B // R,)

    def row_spec(shape):
        return pl.BlockSpec(shape, lambda i: (i, 0))

    def full_spec(shape):
        return pl.BlockSpec(shape, lambda i: (0, 0))

    return pl.pallas_call(
        functools.partial(_dense_body, F),
        grid=grid,
        in_specs=[
            row_spec((R, FD)),
            row_spec((R, F)),
            full_spec(sf.shape),
            full_spec(tf.shape),
            full_spec(cw.shape),
            full_spec(ca.shape),
            full_spec(cc.shape),
            full_spec(E.shape),
            full_spec(w1.shape),
            full_spec(a1.shape),
            full_spec(c1.shape),
            full_spec(w2.shape),
            full_spec(a2.shape),
            full_spec(c2.shape),
            full_spec(w3t.shape),
            full_spec(cb.shape),
        ],
        out_specs=pl.BlockSpec((1, R), lambda i: (0, i)),
        out_shape=jax.ShapeDtypeStruct((1, B), jnp.float32),
        compiler_params=pltpu.CompilerParams(
            dimension_semantics=("parallel",)),
    )(eg2, lv2, sf, tf, cw, ca, cc, E, w1, a1, c1, w2, a2, c2, w3t, cb)


def kernel(x, emb, lin_w, lin_b, bn0_g, bn0_b, ctrl_w, ctrl_b, cbn_g, cbn_b,
           w1, b1, g1, be1, w2, b2, g2, be2, w3, b3):
    B, F = x.shape
    V, D = emb.shape
    per_field = V // F

    offs = (jnp.arange(F, dtype=jnp.int32) * per_field)[None, :]
    idx_flat = (x + offs).reshape(B * F)
    lin2 = lin_w[:, 0].reshape(V // _L, _L)

    n_main = (V // 1024) * 1024
    tail_flat = emb[n_main:, :].reshape((V - n_main) * D)
    table = _sc_transpose(emb.T, tail_flat, V, D).reshape(V, D)

    sf = jnp.repeat(bn0_g * _INV, D)[None, :]
    tf = jnp.repeat(bn0_b, D)[None, :]
    ca = (cbn_g * _INV)[None, :]
    cc = (ctrl_b * cbn_g * _INV + cbn_b)[None, :]
    a1 = (g1 * _INV)[None, :]
    c1 = (b1 * g1 * _INV + be1)[None, :]
    a2 = (g2 * _INV)[None, :]
    c2 = (b2 * g2 * _INV + be2)[None, :]
    E = (jnp.arange(F * D, dtype=jnp.int32)[None, :] // D
         == jnp.arange(F, dtype=jnp.int32)[:, None]).astype(jnp.float32)
    w3t = w3.reshape(1, -1)
    cb = (lin_b + b3).reshape(1, 1)

    halves = []
    H = B // 2
    for h in range(2):
        idx_h = lax.dynamic_slice_in_dim(idx_flat, h * H * F, H * F)
        eg, lvf = _sc_gather(table, lin2, idx_h)
        eg2 = eg.reshape(H, F * D)
        lv2 = lvf.reshape(H, F)
        halves.append(_tc_dense(eg2, lv2, sf, tf, ctrl_w, ca, cc, E,
                                w1, a1, c1, w2, a2, c2, w3t, cb))
    y = jnp.concatenate(halves, axis=1)
    return y.reshape(B)


# speedup vs baseline: 2.5053x; 2.5053x over previous
import dataclasses
import functools

import jax
import jax.numpy as jnp
import numpy as np
from jax import lax
from jax.experimental import pallas as pl
from jax.experimental.pallas import tpu as pltpu
from jax.experimental.pallas import tpu_sc as plsc

_K = 8
_INV = float(1.0 / np.sqrt(1.0 + 1e-5))

_NC = 2
_NS = 16
_L = 16


def _sc_transpose(embT, tail_flat, V, D):
    CK = 1024
    n_full = V // CK
    tail = V - n_full * CK
    rounds = (n_full + _NC * _NS - 1) // (_NC * _NS)

    mesh = plsc.VectorSubcoreMesh(core_axis_name="c", subcore_axis_name="s")
    cp = pltpu.CompilerParams(needs_layout_passes=False,
                              use_tc_tiling_on_sc=True)

    nw = _NC * _NS
    assert rounds % 2 == 0

    @functools.partial(
        pl.kernel,
        mesh=mesh,
        compiler_params=cp,
        out_type=jax.ShapeDtypeStruct((V * D,), jnp.float32),
        scratch_types=[
            pltpu.VMEM((D, CK), jnp.float32),
            pltpu.VMEM((D, CK), jnp.float32),
            pltpu.VMEM((CK * D,), jnp.float32),
            pltpu.VMEM((CK * D,), jnp.float32),
            pltpu.SemaphoreType.DMA,
            pltpu.SemaphoreType.DMA,
            pltpu.SemaphoreType.DMA,
            pltpu.SemaphoreType.DMA,
        ],
    )
    def transpose_kernel(embT_hbm, out_hbm, buf0, buf1, outf0, outf1,
                         si0, si1, so0, so1):
        wid = lax.axis_index("s") * _NC + lax.axis_index("c")
        scat = lax.iota(jnp.int32, _L) * D

        def in_desc(t, buf, sem):
            c0 = (t * nw + wid) * CK
            return pltpu.make_async_copy(
                embT_hbm.at[:, pl.ds(c0, CK)], buf, sem)

        def out_desc(t, outf, sem):
            c0 = (t * nw + wid) * CK
            return pltpu.make_async_copy(
                outf, out_hbm.at[pl.ds(c0 * D, CK * D)], sem)

        def valid(t):
            return (t >= 0) & (t < rounds) & (t * nw + wid < n_full)

        def start_in(t, buf, sem):
            @pl.when(valid(t))
            def _():
                in_desc(t, buf, sem).start()

        def step(t, buf, sem_i, outf, sem_o):
            @pl.when(valid(t - 2))
            def _():
                out_desc(t - 2, outf, sem_o).wait()

            @pl.when(valid(t))
            def _():
                in_desc(t, buf, sem_i).wait()

                @pl.loop(0, CK, step=2 * _L)
                def _cols(l0):
                    for h in range(2):
                        lo = l0 + h * _L
                        for d in range(D):
                            vals = buf[d, pl.ds(lo, _L)]
                            plsc.store_scatter(
                                outf, [scat + (lo * D + d)], vals)

                out_desc(t, outf, sem_o).start()

        start_in(0, buf0, si0)
        start_in(1, buf1, si1)

        @pl.loop(0, rounds, step=2)
        def _round(t):
            start_in(t + 2, buf0, si0)
            step(t, buf0, si0, outf0, so0)
            start_in(t + 3, buf1, si1)
            step(t + 1, buf1, si1, outf1, so1)

        @pl.when(valid(rounds - 2))
        def _():
            out_desc(rounds - 2, outf0, so0).wait()

        @pl.when(valid(rounds - 1))
        def _():
            out_desc(rounds - 1, outf1, so1).wait()

    out = transpose_kernel(embT)
    if tail:
        out = lax.dynamic_update_slice(out, tail_flat, (n_full * CK * D,))
    return out


def _sc_gather(emb, lin2, idx_flat):
    n_idx = idx_flat.shape[0]
    V, D = emb.shape
    nw = _NC * _NS
    per_w = n_idx // nw
    C = 1664
    nchunk = per_w // C
    assert per_w % C == 0

    mesh = plsc.VectorSubcoreMesh(core_axis_name="c", subcore_axis_name="s")
    cp = pltpu.CompilerParams(needs_layout_passes=False,
                              use_tc_tiling_on_sc=False)

    @functools.partial(
        pl.kernel,
        mesh=mesh,
        compiler_params=cp,
        out_type=[
            jax.ShapeDtypeStruct((n_idx, D), jnp.float32),
            jax.ShapeDtypeStruct((n_idx,), jnp.float32),
        ],
        scratch_types=[
            pltpu.VMEM((C,), jnp.int32),
            pltpu.VMEM((C,), jnp.int32),
            pltpu.VMEM((C,), jnp.int32),
            pltpu.VMEM((C, D), jnp.float32),
            pltpu.VMEM((C, _L), jnp.float32),
            pltpu.VMEM((C,), jnp.float32),
            pltpu.SemaphoreType.DMA,
            pltpu.SemaphoreType.DMA,
        ],
    )
    def gather_kernel(emb_hbm, lin_hbm, idx_hbm, eg_hbm, lv_hbm,
                      idx_v, hi_v, lo_v, erows_v, lgran_v, lv_v, sem0, sem1):
        wid = lax.axis_index("s") * _NC + lax.axis_index("c")
        base = wid * per_w

        @pl.loop(0, nchunk)
        def _chunk(k):
            off = base + k * C
            pltpu.sync_copy(idx_hbm.at[pl.ds(off, C)], idx_v)

            @pl.loop(0, C, step=_L)
            def _split(i):
                v = idx_v[pl.ds(i, _L)]
                hi_v[pl.ds(i, _L)] = lax.shift_right_logical(v, 4)
                lo_v[pl.ds(i, _L)] = lax.bitwise_and(v, 15)

            ce = pltpu.async_copy(emb_hbm.at[idx_v], erows_v, sem0)
            cl = pltpu.async_copy(lin_hbm.at[hi_v], lgran_v, sem1)
            ce.wait()
            cl.wait()

            @pl.loop(0, C, step=_L)
            def _select(i):
                rows = lax.iota(jnp.int32, _L) + i
                lv_v[pl.ds(i, _L)] = plsc.load_gather(
                    lgran_v, [rows, lo_v[pl.ds(i, _L)]])

            pltpu.sync_copy(erows_v, eg_hbm.at[pl.ds(off, C)])
            pltpu.sync_copy(lv_v, lv_hbm.at[pl.ds(off, C)])

    return gather_kernel(emb, lin2, idx_flat)


def _dense_body(F, eg_ref, lv_ref, sf_ref, tf_ref, cw_ref, ca_ref, cc_ref,
                E_ref, w1_ref, a1_ref, c1_ref, w2_ref, a2_ref, c2_ref,
                w3_ref, cb_ref, y_ref):
    hi = lax.Precision.DEFAULT
    eb = eg_ref[...] * sf_ref[...] + tf_ref[...]
    z = jnp.dot(eb, cw_ref[...], precision=hi,
                preferred_element_type=jnp.float32)
    u = jnp.maximum(z * ca_ref[...] + cc_ref[...], 0.0)

    ut = u.T
    ii = lax.broadcasted_iota(jnp.int32, ut.shape, 0)
    rank = jnp.zeros(ut.shape, jnp.int32)
    for j in range(F):
        uj = ut[j:j + 1, :]
        beats = (uj > ut) | ((uj == ut) & (j < ii))
        rank = rank + beats.astype(jnp.int32)
    sel = rank < _K

    m = jnp.max(ut, axis=0, keepdims=True)
    p = jnp.where(sel, jnp.exp(ut - m), 0.0)
    s = jnp.sum(p, axis=0, keepdims=True)
    mask = (p / s).T

    me = jnp.dot(mask, E_ref[...], precision=hi,
                 preferred_element_type=jnp.float32)
    em = eb * me
    h1 = jnp.maximum(jnp.dot(em, w1_ref[...], precision=hi,
                             preferred_element_type=jnp.float32)
                     * a1_ref[...] + c1_ref[...], 0.0)
    h2 = jnp.maximum(jnp.dot(h1, w2_ref[...], precision=hi,
                             preferred_element_type=jnp.float32)
                     * a2_ref[...] + c2_ref[...], 0.0)
    out = jnp.sum(h2 * w3_ref[...], axis=1, keepdims=True)
    lin = jnp.sum(lv_ref[...], axis=1, keepdims=True)
    y_ref[...] = jax.nn.sigmoid(out + lin + cb_ref[0, 0]).T


def _tc_dense(eg2, lv2, sf, tf, cw, ca, cc, E, w1, a1, c1, w2, a2, c2, w3t, cb):
    B, FD = eg2.shape
    F = lv2.shape[1]
    R = 512
    grid = (B // R,)

    def row_spec(shape):
        return pl.BlockSpec(shape, lambda i: (i, 0))

    def full_spec(shape):
        return pl.BlockSpec(shape, lambda i: (0, 0))

    return pl.pallas_call(
        functools.partial(_dense_body, F),
        grid=grid,
        in_specs=[
            row_spec((R, FD)),
            row_spec((R, F)),
            full_spec(sf.shape),
            full_spec(tf.shape),
            full_spec(cw.shape),
            full_spec(ca.shape),
            full_spec(cc.shape),
            full_spec(E.shape),
            full_spec(w1.shape),
            full_spec(a1.shape),
            full_spec(c1.shape),
            full_spec(w2.shape),
            full_spec(a2.shape),
            full_spec(c2.shape),
            full_spec(w3t.shape),
            full_spec(cb.shape),
        ],
        out_specs=pl.BlockSpec((1, R), lambda i: (0, i)),
        out_shape=jax.ShapeDtypeStruct((1, B), jnp.float32),
        compiler_params=pltpu.CompilerParams(
            dimension_semantics=("parallel",)),
    )(eg2, lv2, sf, tf, cw, ca, cc, E, w1, a1, c1, w2, a2, c2, w3t, cb)


def kernel(x, emb, lin_w, lin_b, bn0_g, bn0_b, ctrl_w, ctrl_b, cbn_g, cbn_b,
           w1, b1, g1, be1, w2, b2, g2, be2, w3, b3):
    B, F = x.shape
    V, D = emb.shape
    per_field = V // F

    offs = (jnp.arange(F, dtype=jnp.int32) * per_field)[None, :]
    idx_flat = (x + offs).reshape(B * F)
    lin2 = lin_w[:, 0].reshape(V // _L, _L)

    n_main = (V // 1024) * 1024
    tail_flat = emb[n_main:, :].reshape((V - n_main) * D)
    table = _sc_transpose(emb.T, tail_flat, V, D).reshape(V, D)

    sf = jnp.repeat(bn0_g * _INV, D)[None, :]
    tf = jnp.repeat(bn0_b, D)[None, :]
    ca = (cbn_g * _INV)[None, :]
    cc = (ctrl_b * cbn_g * _INV + cbn_b)[None, :]
    a1 = (g1 * _INV)[None, :]
    c1 = (b1 * g1 * _INV + be1)[None, :]
    a2 = (g2 * _INV)[None, :]
    c2 = (b2 * g2 * _INV + be2)[None, :]
    E = (jnp.arange(F * D, dtype=jnp.int32)[None, :] // D
         == jnp.arange(F, dtype=jnp.int32)[:, None]).astype(jnp.float32)
    w3t = w3.reshape(1, -1)
    cb = (lin_b + b3).reshape(1, 1)

    halves = []
    H = B // 2
    for h in range(2):
        idx_h = lax.dynamic_slice_in_dim(idx_flat, h * H * F, H * F)
        eg, lvf = _sc_gather(table, lin2, idx_h)
        eg2 = eg.reshape(H, F * D)
        lv2 = lvf.reshape(H, F)
        halves.append(_tc_dense(eg2, lv2, sf, tf, ctrl_w, ca, cc, E,
                                w1, a1, c1, w2, a2, c2, w3t, cb))
    y = jnp.concatenate(halves, axis=1)
    return y.reshape(B)


# speedup vs baseline: 2.5133x; 1.0032x over previous
import dataclasses
import functools

import jax
import jax.numpy as jnp
import numpy as np
from jax import lax
from jax.experimental import pallas as pl
from jax.experimental.pallas import tpu as pltpu
from jax.experimental.pallas import tpu_sc as plsc

_K = 8
_INV = float(1.0 / np.sqrt(1.0 + 1e-5))

_NC = 2
_NS = 16
_L = 16


def _sc_transpose(embT, tail_flat, V, D):
    CK = 512
    n_full = V // CK
    tail = V - n_full * CK
    rounds = (n_full + _NC * _NS - 1) // (_NC * _NS)
    rounds += rounds % 2

    mesh = plsc.VectorSubcoreMesh(core_axis_name="c", subcore_axis_name="s")
    cp = pltpu.CompilerParams(needs_layout_passes=False,
                              use_tc_tiling_on_sc=True)

    nw = _NC * _NS
    assert rounds % 2 == 0

    @functools.partial(
        pl.kernel,
        mesh=mesh,
        compiler_params=cp,
        out_type=jax.ShapeDtypeStruct((V * D,), jnp.float32),
        scratch_types=[
            pltpu.VMEM((D, CK), jnp.float32),
            pltpu.VMEM((D, CK), jnp.float32),
            pltpu.VMEM((CK * D,), jnp.float32),
            pltpu.VMEM((CK * D,), jnp.float32),
            pltpu.SemaphoreType.DMA,
            pltpu.SemaphoreType.DMA,
            pltpu.SemaphoreType.DMA,
            pltpu.SemaphoreType.DMA,
        ],
    )
    def transpose_kernel(embT_hbm, out_hbm, buf0, buf1, outf0, outf1,
                         si0, si1, so0, so1):
        wid = lax.axis_index("s") * _NC + lax.axis_index("c")
        scat = lax.iota(jnp.int32, _L) * D

        def in_desc(t, buf, sem):
            c0 = (t * nw + wid) * CK
            return pltpu.make_async_copy(
                embT_hbm.at[:, pl.ds(c0, CK)], buf, sem)

        def out_desc(t, outf, sem):
            c0 = (t * nw + wid) * CK
            return pltpu.make_async_copy(
                outf, out_hbm.at[pl.ds(c0 * D, CK * D)], sem)

        def valid(t):
            return (t >= 0) & (t < rounds) & (t * nw + wid < n_full)

        def start_in(t, buf, sem):
            @pl.when(valid(t))
            def _():
                in_desc(t, buf, sem).start()

        def step(t, buf, sem_i, outf, sem_o):
            @pl.when(valid(t - 2))
            def _():
                out_desc(t - 2, outf, sem_o).wait()

            @pl.when(valid(t))
            def _():
                in_desc(t, buf, sem_i).wait()

                @pl.loop(0, CK, step=2 * _L)
                def _cols(l0):
                    for h in range(2):
                        lo = l0 + h * _L
                        for d in range(D):
                            vals = buf[d, pl.ds(lo, _L)]
                            plsc.store_scatter(
                                outf, [scat + (lo * D + d)], vals)

                out_desc(t, outf, sem_o).start()

        start_in(0, buf0, si0)
        start_in(1, buf1, si1)

        @pl.loop(0, rounds, step=2)
        def _round(t):
            start_in(t + 2, buf0, si0)
            step(t, buf0, si0, outf0, so0)
            start_in(t + 3, buf1, si1)
            step(t + 1, buf1, si1, outf1, so1)

        @pl.when(valid(rounds - 2))
        def _():
            out_desc(rounds - 2, outf0, so0).wait()

        @pl.when(valid(rounds - 1))
        def _():
            out_desc(rounds - 1, outf1, so1).wait()

    out = transpose_kernel(embT)
    if tail:
        out = lax.dynamic_update_slice(out, tail_flat, (n_full * CK * D,))
    return out


def _sc_gather(emb, lin2, idx_flat):
    n_idx = idx_flat.shape[0]
    V, D = emb.shape
    nw = _NC * _NS
    per_w = n_idx // nw
    C = 1664
    nchunk = per_w // C
    assert per_w % C == 0

    mesh = plsc.VectorSubcoreMesh(core_axis_name="c", subcore_axis_name="s")
    cp = pltpu.CompilerParams(needs_layout_passes=False,
                              use_tc_tiling_on_sc=False)

    @functools.partial(
        pl.kernel,
        mesh=mesh,
        compiler_params=cp,
        out_type=[
            jax.ShapeDtypeStruct((n_idx, D), jnp.float32),
            jax.ShapeDtypeStruct((n_idx,), jnp.float32),
        ],
        scratch_types=[
            pltpu.VMEM((C,), jnp.int32),
            pltpu.VMEM((C,), jnp.int32),
            pltpu.VMEM((C,), jnp.int32),
            pltpu.VMEM((C, D), jnp.float32),
            pltpu.VMEM((C, _L), jnp.float32),
            pltpu.VMEM((C,), jnp.float32),
            pltpu.SemaphoreType.DMA,
            pltpu.SemaphoreType.DMA,
        ],
    )
    def gather_kernel(emb_hbm, lin_hbm, idx_hbm, eg_hbm, lv_hbm,
                      idx_v, hi_v, lo_v, erows_v, lgran_v, lv_v, sem0, sem1):
        wid = lax.axis_index("s") * _NC + lax.axis_index("c")
        base = wid * per_w

        @pl.loop(0, nchunk)
        def _chunk(k):
            off = base + k * C
            pltpu.sync_copy(idx_hbm.at[pl.ds(off, C)], idx_v)

            @pl.loop(0, C, step=_L)
            def _split(i):
                v = idx_v[pl.ds(i, _L)]
                hi_v[pl.ds(i, _L)] = lax.shift_right_logical(v, 4)
                lo_v[pl.ds(i, _L)] = lax.bitwise_and(v, 15)

            ce = pltpu.async_copy(emb_hbm.at[idx_v], erows_v, sem0)
            cl = pltpu.async_copy(lin_hbm.at[hi_v], lgran_v, sem1)
            ce.wait()
            cl.wait()

            @pl.loop(0, C, step=_L)
            def _select(i):
                rows = lax.iota(jnp.int32, _L) + i
                lv_v[pl.ds(i, _L)] = plsc.load_gather(
                    lgran_v, [rows, lo_v[pl.ds(i, _L)]])

            pltpu.sync_copy(erows_v, eg_hbm.at[pl.ds(off, C)])
            pltpu.sync_copy(lv_v, lv_hbm.at[pl.ds(off, C)])

    return gather_kernel(emb, lin2, idx_flat)


def _dense_body(F, eg_ref, lv_ref, sf_ref, tf_ref, cw_ref, ca_ref, cc_ref,
                E_ref, w1_ref, a1_ref, c1_ref, w2_ref, a2_ref, c2_ref,
                w3_ref, cb_ref, y_ref):
    hi = lax.Precision.DEFAULT
    eb = eg_ref[...] * sf_ref[...] + tf_ref[...]
    z = jnp.dot(eb, cw_ref[...], precision=hi,
                preferred_element_type=jnp.float32)
    u = jnp.maximum(z * ca_ref[...] + cc_ref[...], 0.0)

    ut = u.T
    ii = lax.broadcasted_iota(jnp.int32, ut.shape, 0)
    rank = jnp.zeros(ut.shape, jnp.int32)
    for j in range(F):
        uj = ut[j:j + 1, :]
        beats = (uj > ut) | ((uj == ut) & (j < ii))
        rank = rank + beats.astype(jnp.int32)
    sel = rank < _K

    m = jnp.max(ut, axis=0, keepdims=True)
    p = jnp.where(sel, jnp.exp(ut - m), 0.0)
    s = jnp.sum(p, axis=0, keepdims=True)
    mask = (p / s).T

    me = jnp.dot(mask, E_ref[...], precision=hi,
                 preferred_element_type=jnp.float32)
    em = eb * me
    h1 = jnp.maximum(jnp.dot(em, w1_ref[...], precision=hi,
                             preferred_element_type=jnp.float32)
                     * a1_ref[...] + c1_ref[...], 0.0)
    h2 = jnp.maximum(jnp.dot(h1, w2_ref[...], precision=hi,
                             preferred_element_type=jnp.float32)
                     * a2_ref[...] + c2_ref[...], 0.0)
    out = jnp.sum(h2 * w3_ref[...], axis=1, keepdims=True)
    lin = jnp.sum(lv_ref[...], axis=1, keepdims=True)
    y_ref[...] = jax.nn.sigmoid(out + lin + cb_ref[0, 0]).T


def _tc_dense(eg2, lv2, sf, tf, cw, ca, cc, E, w1, a1, c1, w2, a2, c2, w3t, cb):
    B, FD = eg2.shape
    F = lv2.shape[1]
    R = 512
    grid = (B // R,)

    def row_spec(shape):
        return pl.BlockSpec(shape, lambda i: (i, 0))

    def full_spec(shape):
        return pl.BlockSpec(shape, lambda i: (0, 0))

    return pl.pallas_call(
        functools.partial(_dense_body, F),
        grid=grid,
        in_specs=[
            row_spec((R, FD)),
            row_spec((R, F)),
            full_spec(sf.shape),
            full_spec(tf.shape),
            full_spec(cw.shape),
            full_spec(ca.shape),
            full_spec(cc.shape),
            full_spec(E.shape),
            full_spec(w1.shape),
            full_spec(a1.shape),
            full_spec(c1.shape),
            full_spec(w2.shape),
            full_spec(a2.shape),
            full_spec(c2.shape),
            full_spec(w3t.shape),
            full_spec(cb.shape),
        ],
        out_specs=pl.BlockSpec((1, R), lambda i: (0, i)),
        out_shape=jax.ShapeDtypeStruct((1, B), jnp.float32),
        compiler_params=pltpu.CompilerParams(
            dimension_semantics=("parallel",)),
    )(eg2, lv2, sf, tf, cw, ca, cc, E, w1, a1, c1, w2, a2, c2, w3t, cb)


def kernel(x, emb, lin_w, lin_b, bn0_g, bn0_b, ctrl_w, ctrl_b, cbn_g, cbn_b,
           w1, b1, g1, be1, w2, b2, g2, be2, w3, b3):
    B, F = x.shape
    V, D = emb.shape
    per_field = V // F

    offs = (jnp.arange(F, dtype=jnp.int32) * per_field)[None, :]
    idx_flat = (x + offs).reshape(B * F)
    lin2 = lin_w[:, 0].reshape(V // _L, _L)

    n_main = (V // 1024) * 1024
    tail_flat = emb[n_main:, :].reshape((V - n_main) * D)
    table = _sc_transpose(emb.T, tail_flat, V, D).reshape(V, D)

    sf = jnp.repeat(bn0_g * _INV, D)[None, :]
    tf = jnp.repeat(bn0_b, D)[None, :]
    ca = (cbn_g * _INV)[None, :]
    cc = (ctrl_b * cbn_g * _INV + cbn_b)[None, :]
    a1 = (g1 * _INV)[None, :]
    c1 = (b1 * g1 * _INV + be1)[None, :]
    a2 = (g2 * _INV)[None, :]
    c2 = (b2 * g2 * _INV + be2)[None, :]
    E = (jnp.arange(F * D, dtype=jnp.int32)[None, :] // D
         == jnp.arange(F, dtype=jnp.int32)[:, None]).astype(jnp.float32)
    w3t = w3.reshape(1, -1)
    cb = (lin_b + b3).reshape(1, 1)

    halves = []
    H = B // 2
    for h in range(2):
        idx_h = lax.dynamic_slice_in_dim(idx_flat, h * H * F, H * F)
        eg, lvf = _sc_gather(table, lin2, idx_h)
        eg2 = eg.reshape(H, F * D)
        lv2 = lvf.reshape(H, F)
        halves.append(_tc_dense(eg2, lv2, sf, tf, ctrl_w, ca, cc, E,
                                w1, a1, c1, w2, a2, c2, w3t, cb))
    y = jnp.concatenate(halves, axis=1)
    return y.reshape(B)
